# Initial kernel scaffold; baseline (speedup 1.0000x reference)
import jax
import jax.numpy as jnp
from jax.experimental import pallas as pl


def kernel(x, embed, adjs, W_ie, b_ie, W_is, b_is, W_iem, b_iem, W_ce, b_ce, W_cs, b_cs, W_cem, b_cem, W_o11, b_o11, W_o111, b_o111, W_o12, b_o12, g_ie, be_ie, g_is, be_is, g_iem, be_iem, g_ce, be_ce, g_cs, be_cs, g_cem, be_cem, g_o1, be_o1, g_o111, be_o111):
    raise NotImplementedError("write your pallas kernel here")


# speedup vs baseline: 1.0202x; 1.0202x over previous
import functools

import jax
import jax.numpy as jnp
from jax.experimental import pallas as pl
from jax.experimental.pallas import tpu as pltpu

EPS = 1e-5


def _elu(v):
    return jnp.where(v > 0, v, jnp.exp(jnp.minimum(v, 0.0)) - 1.0)


def _support_kernel(x_ref, w_ref, o_ref):
    o_ref[0] = jnp.dot(x_ref[0], w_ref[0], preferred_element_type=jnp.float32)


def _spmm_kernel(adj_ref, sup_ref, b_ref, o_ref, st_ref):
    m = pl.program_id(1)
    h = jnp.dot(adj_ref[0], sup_ref[0], preferred_element_type=jnp.float32)
    h = h + b_ref[0]
    o_ref[0] = h
    s0 = jnp.sum(h, axis=0, keepdims=True)
    s1 = jnp.sum(h * h, axis=0, keepdims=True)
    blk = jnp.concatenate(
        [s0, s1, jnp.zeros((6, h.shape[1]), jnp.float32)], axis=0)

    @pl.when(m == 0)
    def _():
        st_ref[0] = blk

    @pl.when(m != 0)
    def _():
        st_ref[0] = st_ref[0] + blk


def _bn_scale_shift(st_row0, st_row1, g, be, n_rows):
    mean = st_row0 / n_rows
    var = st_row1 / n_rows - mean * mean
    scale = g / jnp.sqrt(var + EPS)
    shift = be - mean * scale
    return scale, shift


def _mid_kernel(n_rows, h_ref, st_ref, g_ref, be_ref, w_ref, o_ref):
    scale, shift = _bn_scale_shift(
        st_ref[0, 0:1, :], st_ref[0, 1:2, :], g_ref[0], be_ref[0], n_rows)
    a = _elu(h_ref[0] * scale + shift)
    o_ref[0] = jnp.dot(a, w_ref[0], preferred_element_type=jnp.float32)


def _head1_kernel(n_rows, h_ref, st_ref, g_ref, be_ref, w_ref, b_ref,
                  o_ref, so_ref):
    m = pl.program_id(0)
    hdim = w_ref.shape[1]
    acc = jnp.broadcast_to(b_ref[...], (h_ref.shape[1], hdim)).astype(
        jnp.float32)
    for s in range(3):
        scale, shift = _bn_scale_shift(
            st_ref[s, 0:1, :], st_ref[s, 1:2, :], g_ref[s], be_ref[s], n_rows)
        a = _elu(h_ref[s] * scale + shift)
        acc = acc + jnp.dot(a, w_ref[s * a.shape[1]:(s + 1) * a.shape[1], :],
                            preferred_element_type=jnp.float32)
    o_ref[...] = acc
    s0 = jnp.sum(acc, axis=0, keepdims=True)
    s1 = jnp.sum(acc * acc, axis=0, keepdims=True)
    blk = jnp.concatenate([s0, s1, jnp.zeros((6, hdim), jnp.float32)], axis=0)

    @pl.when(m == 0)
    def _():
        so_ref[...] = blk

    @pl.when(m != 0)
    def _():
        so_ref[...] = so_ref[...] + blk


def _head2_kernel(n_rows, t_ref, st_ref, g_ref, be_ref, w_ref, b_ref,
                  o_ref, so_ref):
    m = pl.program_id(0)
    scale, shift = _bn_scale_shift(
        st_ref[0:1, :], st_ref[1:2, :], g_ref[...], be_ref[...], n_rows)
    a = _elu(t_ref[...] * scale + shift)
    t = jnp.dot(a, w_ref[...], preferred_element_type=jnp.float32) + b_ref[...]
    o_ref[...] = t
    s0 = jnp.sum(t, axis=0, keepdims=True)
    s1 = jnp.sum(t * t, axis=0, keepdims=True)
    blk = jnp.concatenate(
        [s0, s1, jnp.zeros((6, t.shape[1]), jnp.float32)], axis=0)

    @pl.when(m == 0)
    def _():
        so_ref[...] = blk

    @pl.when(m != 0)
    def _():
        so_ref[...] = so_ref[...] + blk


def _out_kernel(n_rows, t_ref, st_ref, g_ref, be_ref, w_ref, b_ref, o_ref):
    scale, shift = _bn_scale_shift(
        st_ref[0:1, :], st_ref[1:2, :], g_ref[...], be_ref[...], n_rows)
    a = _elu(t_ref[...] * scale + shift)
    logits = jnp.dot(a, w_ref[...],
                     preferred_element_type=jnp.float32) + b_ref[...]
    mx = jnp.max(logits, axis=1, keepdims=True)
    sh = logits - mx
    lse = jnp.log(jnp.sum(jnp.exp(sh), axis=1, keepdims=True))
    o_ref[...] = sh - lse


def kernel(x, embed, adjs, W_ie, b_ie, W_is, b_is, W_iem, b_iem, W_ce, b_ce,
           W_cs, b_cs, W_cem, b_cem, W_o11, b_o11, W_o111, b_o111, W_o12,
           b_o12, g_ie, be_ie, g_is, be_is, g_iem, be_iem, g_ce, be_ce, g_cs,
           be_cs, g_cem, be_cem, g_o1, be_o1, g_o111, be_o111):
    n, f = x.shape
    hdim = W_ie.shape[1]
    odim = W_o12.shape[1]
    fn = float(n)

    bm = min(400, n)
    bms = min(1000, n)
    nb = n // bm
    nbs = n // bms

    xin = jnp.stack([x, x, embed])
    w_in = jnp.stack([W_ie, W_is, W_iem])
    b_in = jnp.stack([b_ie, b_is, b_iem])[:, None, :]
    w_c = jnp.stack([W_ce, W_cs, W_cem])
    b_c = jnp.stack([b_ce, b_cs, b_cem])[:, None, :]
    g_i = jnp.stack([g_ie, g_is, g_iem])[:, None, :]
    be_i = jnp.stack([be_ie, be_is, be_iem])[:, None, :]
    g_c = jnp.stack([g_ce, g_cs, g_cem])[:, None, :]
    be_c = jnp.stack([be_ce, be_cs, be_cem])[:, None, :]

    f32 = jnp.float32

    sup1 = pl.pallas_call(
        _support_kernel,
        grid=(3,),
        in_specs=[
            pl.BlockSpec((1, n, f), lambda s: (s, 0, 0)),
            pl.BlockSpec((1, f, hdim), lambda s: (s, 0, 0)),
        ],
        out_specs=pl.BlockSpec((1, n, hdim), lambda s: (s, 0, 0)),
        out_shape=jax.ShapeDtypeStruct((3, n, hdim), f32),
    )(xin, w_in)

    def spmm(sup, bias):
        return pl.pallas_call(
            _spmm_kernel,
            grid=(3, nb),
            in_specs=[
                pl.BlockSpec((1, bm, n), lambda s, m: (s, m, 0)),
                pl.BlockSpec((1, n, hdim), lambda s, m: (s, 0, 0)),
                pl.BlockSpec((1, 1, hdim), lambda s, m: (s, 0, 0)),
            ],
            out_specs=[
                pl.BlockSpec((1, bm, hdim), lambda s, m: (s, m, 0)),
                pl.BlockSpec((1, 8, hdim), lambda s, m: (s, 0, 0)),
            ],
            out_shape=[
                jax.ShapeDtypeStruct((3, n, hdim), f32),
                jax.ShapeDtypeStruct((3, 8, hdim), f32),
            ],
            compiler_params=pltpu.CompilerParams(
                dimension_semantics=("arbitrary", "arbitrary")),
        )(adjs, sup, bias)

    h1, st1 = spmm(sup1, b_in)

    sup2 = pl.pallas_call(
        functools.partial(_mid_kernel, fn),
        grid=(3, nbs),
        in_specs=[
            pl.BlockSpec((1, bms, hdim), lambda s, m: (s, m, 0)),
            pl.BlockSpec((1, 8, hdim), lambda s, m: (s, 0, 0)),
            pl.BlockSpec((1, 1, hdim), lambda s, m: (s, 0, 0)),
            pl.BlockSpec((1, 1, hdim), lambda s, m: (s, 0, 0)),
            pl.BlockSpec((1, hdim, hdim), lambda s, m: (s, 0, 0)),
        ],
        out_specs=pl.BlockSpec((1, bms, hdim), lambda s, m: (s, m, 0)),
        out_shape=jax.ShapeDtypeStruct((3, n, hdim), f32),
    )(h1, st1, g_i, be_i, w_c)

    h2, st2 = spmm(sup2, b_c)

    t1, stt1 = pl.pallas_call(
        functools.partial(_head1_kernel, fn),
        grid=(nbs,),
        in_specs=[
            pl.BlockSpec((3, bms, hdim), lambda m: (0, m, 0)),
            pl.BlockSpec((3, 8, hdim), lambda m: (0, 0, 0)),
            pl.BlockSpec((3, 1, hdim), lambda m: (0, 0, 0)),
            pl.BlockSpec((3, 1, hdim), lambda m: (0, 0, 0)),
            pl.BlockSpec((3 * hdim, hdim), lambda m: (0, 0)),
            pl.BlockSpec((1, hdim), lambda m: (0, 0)),
        ],
        out_specs=[
            pl.BlockSpec((bms, hdim), lambda m: (m, 0)),
            pl.BlockSpec((8, hdim), lambda m: (0, 0)),
        ],
        out_shape=[
            jax.ShapeDtypeStruct((n, hdim), f32),
            jax.ShapeDtypeStruct((8, hdim), f32),
        ],
        compiler_params=pltpu.CompilerParams(
            dimension_semantics=("arbitrary",)),
    )(h2, st2, g_c, be_c, W_o11, b_o11[None, :])

    t2, stt2 = pl.pallas_call(
        functools.partial(_head2_kernel, fn),
        grid=(nbs,),
        in_specs=[
            pl.BlockSpec((bms, hdim), lambda m: (m, 0)),
            pl.BlockSpec((8, hdim), lambda m: (0, 0)),
            pl.BlockSpec((1, hdim), lambda m: (0, 0)),
            pl.BlockSpec((1, hdim), lambda m: (0, 0)),
            pl.BlockSpec((hdim, hdim), lambda m: (0, 0)),
            pl.BlockSpec((1, hdim), lambda m: (0, 0)),
        ],
        out_specs=[
            pl.BlockSpec((bms, hdim), lambda m: (m, 0)),
            pl.BlockSpec((8, hdim), lambda m: (0, 0)),
        ],
        out_shape=[
            jax.ShapeDtypeStruct((n, hdim), f32),
            jax.ShapeDtypeStruct((8, hdim), f32),
        ],
        compiler_params=pltpu.CompilerParams(
            dimension_semantics=("arbitrary",)),
    )(t1, stt1, g_o1[None, :], be_o1[None, :], W_o111, b_o111[None, :])

    out = pl.pallas_call(
        functools.partial(_out_kernel, fn),
        grid=(nbs,),
        in_specs=[
            pl.BlockSpec((bms, hdim), lambda m: (m, 0)),
            pl.BlockSpec((8, hdim), lambda m: (0, 0)),
            pl.BlockSpec((1, hdim), lambda m: (0, 0)),
            pl.BlockSpec((1, hdim), lambda m: (0, 0)),
            pl.BlockSpec((hdim, odim), lambda m: (0, 0)),
            pl.BlockSpec((1, odim), lambda m: (0, 0)),
        ],
        out_specs=pl.BlockSpec((bms, odim), lambda m: (m, 0)),
        out_shape=jax.ShapeDtypeStruct((n, odim), f32),
    )(t2, stt2, g_o111[None, :], be_o111[None, :], W_o12, b_o12[None, :])

    return out
